# baseline (speedup 1.0000x reference)
import functools

import jax
import jax.numpy as jnp
from jax import lax
from jax.experimental import pallas as pl
from jax.experimental.pallas import tpu as pltpu
from jax.experimental.pallas import tpu_sc as plsc

_B, _M, _FIN, _FOUT, _K = 4, 10000, 128, 128, 4
_E = 320000

_NC, _NS, _L = 2, 16, 16
_NB = 80
_NBATCH = 252
_EPAD = _NS * _NBATCH * _NB
_NSLOT = 4
_NQUAD = _NBATCH // _NSLOT
_RBLK = 40
_NRB = _M // _RBLK
_RITER = (_NRB + _NS - 1) // _NS
_CPB = _B // _NC


def _sc_body(x0, rc3, lap3, g1, g2, g3, acc,
             idx_a, idx_b, idx_c, idx_d,
             lapr_a, lapr_b, lapr_c, lapr_d,
             rowv_a, rowv_b, rowv_c, rowv_d,
             cofv_a, cofv_b, cofv_c, cofv_d,
             lapv_a, lapv_b, lapv_c, lapv_d,
             rows_a, rows_b, rows_c, rows_d, zerov,
             isa, isb, isc, isd, gsa, gsb, gsc, gsd, ssa, ssb, ssc, ssd):
    c = lax.axis_index("c")
    s = lax.axis_index("s")
    zero16 = jnp.zeros((_L,), jnp.float32)

    slots = [
        (idx_a, lapr_a, rowv_a, cofv_a, lapv_a, rows_a, isa, gsa, ssa),
        (idx_b, lapr_b, rowv_b, cofv_b, lapv_b, rows_b, isb, gsb, ssb),
        (idx_c, lapr_c, rowv_c, cofv_c, lapv_c, rows_c, isc, gsc, ssc),
        (idx_d, lapr_d, rowv_d, cofv_d, lapv_d, rows_d, isd, gsd, ssd),
    ]

    def _zb(r, carry):
        for j in range(_FIN // _L):
            zerov[r, pl.ds(j * _L, _L)] = zero16
        return carry
    lax.fori_loop(0, _RBLK, _zb, 0)
    for r in range(_RITER):
        blk = s + r * _NS

        @pl.when(blk < _NRB)
        def _():
            pltpu.sync_copy(zerov, acc.at[pl.ds(blk * _RBLK, _RBLK)])
    plsc.subcore_barrier()

    def _issue_idx(x, i):
        idx, lapr, _, _, _, _, isx, _, _ = slots[x]
        pltpu.async_copy(rc3.at[s, i], idx, isx)
        pltpu.async_copy(lap3.at[s, i], lapr, isx)

    def _wait_idx(x):
        idx, lapr, _, _, _, _, isx, _, _ = slots[x]
        pltpu.make_async_copy(rc3.at[s, 0], idx, isx).wait()
        pltpu.make_async_copy(lap3.at[s, 0], lapr, isx).wait()

    def _unpack(x, boff):
        idx, lapr, rowv, cofv, lapv, _, _, _, _ = slots[x]
        for j in range(_NB // _L):
            sl = pl.ds(j * _L, _L)
            v = idx[sl]
            rowv[sl] = lax.shift_right_logical(v, 16)
            cofv[sl] = jnp.bitwise_and(v, 0xFFFF) + boff
            lapv[sl] = lapr[sl]

    def _issue_gather(x, src, i):
        _, _, _, cofv, _, rows, _, gsx, _ = slots[x]
        del i
        pltpu.async_copy(src.at[cofv], rows, gsx)

    def _scale(x):
        _, _, _, _, lapv, rows, _, _, _ = slots[x]

        def _grp(g, carry):
            lap16 = lapv[pl.ds(g * _L, _L)]
            for t in range(_L):
                lv = lap16[t]
                e = g * _L + t
                for j in range(_FIN // _L):
                    sl = pl.ds(j * _L, _L)
                    rows[e, sl] = rows[e, sl] * lv
            return carry
        lax.fori_loop(0, _NB // _L, _grp, 0)

    def _comp(x, src):
        _, _, rowv, _, _, rows, _, gsx, ssx = slots[x]
        pltpu.make_async_copy(src.at[pl.ds(0, _NB)], rows, gsx).wait()
        _scale(x)
        pltpu.async_copy(rows, acc.at[rowv], ssx, add=True)

    def _wait_scatter(x):
        _, _, _, _, _, rows, _, _, ssx = slots[x]
        pltpu.make_async_copy(rows, acc.at[pl.ds(0, _NB)], ssx).wait()

    def _pass(src, dst, boff):
        for x in range(_NSLOT):
            _issue_idx(x, x)
        for x in range(_NSLOT):
            _wait_idx(x)
            _unpack(x, boff)
            _issue_gather(x, src, x)
        for x in range(_NSLOT):
            _issue_idx(x, x + _NSLOT)

        def _quad(ii, carry):
            i0 = _NSLOT * ii

            def _prep(x):
                @pl.when(ii < _NQUAD - 1)
                def _():
                    _wait_idx(x)
                    _wait_scatter(x)
                    _unpack(x, boff)
                    _issue_gather(x, src, i0 + _NSLOT + x)

                    @pl.when(ii < _NQUAD - 2)
                    def _():
                        _issue_idx(x, i0 + 2 * _NSLOT + x)

            _comp(0, src)
            _comp(1, src)
            _prep(0)
            _comp(2, src)
            _prep(1)
            _comp(3, src)
            _prep(2)
            _prep(3)
            return carry
        lax.fori_loop(0, _NQUAD, _quad, 0)
        for x in range(_NSLOT):
            _wait_scatter(x)
        plsc.subcore_barrier()

        for r in range(_RITER):
            blk = s + r * _NS

            @pl.when(blk < _NRB)
            def _():
                r0 = blk * _RBLK
                pltpu.sync_copy(acc.at[pl.ds(r0, _RBLK)],
                                dst.at[pl.ds(boff + r0, _RBLK)])
                pltpu.sync_copy(zerov, acc.at[pl.ds(r0, _RBLK)])
        plsc.subcore_barrier()

    def _chunk(bi, carry):
        boff = (c * _CPB + bi) * _M
        _pass(x0, g1, boff)
        _pass(g1, g2, boff)
        _pass(g2, g3, boff)
        return carry
    lax.fori_loop(0, _CPB, _chunk, 0)


_spmm3 = pl.kernel(
    _sc_body,
    out_type=[jax.ShapeDtypeStruct((_B * _M, _FIN), jnp.float32)] * 3,
    mesh=plsc.VectorSubcoreMesh(core_axis_name="c", subcore_axis_name="s",
                                num_cores=_NC, num_subcores=_NS),
    scratch_types=[
        pltpu.VMEM_SHARED((_M, _FIN), jnp.float32),
    ] + [pltpu.VMEM((_NB,), jnp.int32) for _ in range(4)]
      + [pltpu.VMEM((_NB,), jnp.float32) for _ in range(4)]
      + [pltpu.VMEM((_NB,), jnp.int32) for _ in range(4)]
      + [pltpu.VMEM((_NB,), jnp.int32) for _ in range(4)]
      + [pltpu.VMEM((_NB,), jnp.float32) for _ in range(4)]
      + [pltpu.VMEM((_NB, _FIN), jnp.float32) for _ in range(4)]
      + [pltpu.VMEM((_RBLK, _FIN), jnp.float32)]
      + [pltpu.SemaphoreType.DMA for _ in range(12)],
)


_BMB = 2000
_NBM = _B * _M // _BMB


def _tc_body(x0, g1, g2, g3, w, bias, out):
    acc = jnp.dot(x0[...], w[:, 0, :], preferred_element_type=jnp.float32)
    acc += jnp.dot(g1[...], w[:, 1, :], preferred_element_type=jnp.float32)
    acc += jnp.dot(g2[...], w[:, 2, :], preferred_element_type=jnp.float32)
    acc += jnp.dot(g3[...], w[:, 3, :], preferred_element_type=jnp.float32)
    out[...] = acc + bias[0, 0, :]


_cheb_out = pl.pallas_call(
    _tc_body,
    grid=(_NBM,),
    in_specs=[
        pl.BlockSpec((_BMB, _FIN), lambda i: (i, 0)),
        pl.BlockSpec((_BMB, _FIN), lambda i: (i, 0)),
        pl.BlockSpec((_BMB, _FIN), lambda i: (i, 0)),
        pl.BlockSpec((_BMB, _FIN), lambda i: (i, 0)),
        pl.BlockSpec((_FIN, _K, _FOUT), lambda i: (0, 0, 0)),
        pl.BlockSpec((1, 1, _FOUT), lambda i: (0, 0, 0)),
    ],
    out_specs=pl.BlockSpec((_BMB, _FOUT), lambda i: (i, 0)),
    out_shape=jax.ShapeDtypeStruct((_B * _M, _FOUT), jnp.float32),
)


def kernel(inputs, edge_index, lap_vals, W, b):
    x0 = inputs.reshape(_B * _M, _FIN)
    rc = jnp.left_shift(edge_index[0], 16) | edge_index[1]
    pad = _EPAD - _E
    rc3 = jnp.concatenate(
        [rc, jnp.zeros((pad,), jnp.int32)]).reshape(_NS, _NBATCH, _NB)
    lap3 = jnp.concatenate(
        [lap_vals, jnp.zeros((pad,), jnp.float32)]).reshape(_NS, _NBATCH, _NB)
    g1, g2, g3 = _spmm3(x0, rc3, lap3)
    wt = jnp.stack([W[:, 0, :] - W[:, 2, :],
                    W[:, 1, :] - 3.0 * W[:, 3, :],
                    2.0 * W[:, 2, :],
                    4.0 * W[:, 3, :]], axis=1)
    out = _cheb_out(x0, g1, g2, g3, wt, b)
    return out.reshape(_B, _M, _FOUT)
